# baseline (speedup 1.0000x reference)
import functools

import jax
import jax.numpy as jnp
from jax import lax
from jax.experimental import pallas as pl
from jax.experimental.pallas import tpu as pltpu
from jax.experimental.pallas import tpu_sc as plsc

_MAXR = 128
_ROWS = 2 * _MAXR + 1
_D = 1024
_B = 32 * 2048
_NC, _NS = 2, 16
_NW = _NC * _NS
_NG = 4
_ND = 8
_DCOL = _D // _ND
_RPG = _B // _NG
_RC = 128
_NBUF = 4
_LANES = 16


def _emb_body(idx_hbm, table_hbm, out_hbm, ttile, bufs, idx_v, *sems):
    tsem = sems[0]
    wsem = sems[1:]
    wid = lax.axis_index("s") * _NC + lax.axis_index("c")
    g = wid // _ND
    j = wid % _ND
    rbase = g * _RPG
    cbase = j * _DCOL

    isem = wsem[0]
    pltpu.async_copy(idx_hbm.at[pl.ds(rbase, _RPG)], idx_v, isem)

    def tload_start(v, carry):
        pltpu.async_copy(
            table_hbm.at[pl.ds(v * _D + cbase, _DCOL)],
            ttile.at[pl.ds(v * _DCOL, _DCOL)],
            tsem,
        )
        return carry

    lax.fori_loop(0, _ROWS, tload_start, 0)

    pltpu.make_async_copy(idx_hbm.at[pl.ds(rbase, _RPG)], idx_v, isem).wait()

    def clamp_body(i, carry):
        sl = pl.ds(i * _LANES, _LANES)
        v = idx_v[sl]
        idx_v[sl] = (
            jnp.minimum(jnp.maximum(v, -_MAXR), _MAXR) + _MAXR
        ) * _DCOL
        return carry

    lax.fori_loop(0, _RPG // _LANES, clamp_body, 0)

    def tload_wait(v, carry):
        pltpu.make_async_copy(
            table_hbm.at[pl.ds(v * _D + cbase, _DCOL)],
            ttile.at[pl.ds(v * _DCOL, _DCOL)],
            tsem,
        ).wait()
        return carry

    lax.fori_loop(0, _ROWS, tload_wait, 0)

    colvec = [lax.iota(jnp.int32, _LANES) + k * _LANES for k in range(_ND)]
    lanevec = [jnp.full((_LANES,), u, jnp.int32) for u in range(_LANES)]
    _NCHW = _RPG // _RC
    _BLK = _RC // _LANES

    def outer_body(o, carry):
        for w in range(_NBUF):
            n = o * _NBUF + w
            row0 = rbase + n * _RC

            @pl.when(o > 0)
            def _wait_prev():
                pltpu.make_async_copy(
                    bufs.at[w],
                    out_hbm.at[pl.ds(row0 - _NBUF * _RC, _RC), pl.ds(cbase, _DCOL)],
                    wsem[w],
                ).wait()

            @plsc.parallel_loop(0, 0, 1, unroll=4)
            def _rows(r):
                blk = r & ~(_LANES - 1)
                lane = r & (_LANES - 1)
                v16 = idx_v[pl.ds(n * _RC + blk, _LANES)]
                base = v16.at[jnp.full((_LANES,), lane, jnp.int32)].get(
                    mode="promise_in_bounds"
                )
                for k in range(_ND):
                    val = plsc.load_gather(ttile, [base + colvec[k]])
                    bufs[w, r, pl.ds(k * _LANES, _LANES)] = val

            pltpu.async_copy(
                bufs.at[w],
                out_hbm.at[pl.ds(row0, _RC), pl.ds(cbase, _DCOL)],
                wsem[w],
            )
        return carry

    lax.fori_loop(0, _NCHW // _NBUF, outer_body, 0)

    for w in range(_NBUF):
        row0 = rbase + ((_NCHW // _NBUF - 1) * _NBUF + w) * _RC
        pltpu.make_async_copy(
            bufs.at[w],
            out_hbm.at[pl.ds(row0, _RC), pl.ds(cbase, _DCOL)],
            wsem[w],
        ).wait()


@jax.jit
def _emb_call(idx_flat, table_flat):
    mesh = plsc.VectorSubcoreMesh(core_axis_name="c", subcore_axis_name="s")
    fn = functools.partial(
        pl.kernel,
        mesh=mesh,
        compiler_params=pltpu.CompilerParams(needs_layout_passes=False),
        out_type=jax.ShapeDtypeStruct((_B, _D), jnp.float32),
        scratch_types=[
            pltpu.VMEM((_ROWS * _DCOL,), jnp.float32),
            pltpu.VMEM((_NBUF, _RC, _DCOL), jnp.float32),
            pltpu.VMEM((_RPG,), jnp.int32),
        ]
        + [pltpu.SemaphoreType.DMA] * (1 + _NBUF),
    )(_emb_body)
    return fn(idx_flat, table_flat)


def kernel(relative_dis, W):
    idx_flat = relative_dis.reshape(-1).astype(jnp.int32)
    out = _emb_call(idx_flat, W.reshape(-1))
    return out.reshape(relative_dis.shape + (_D,))


# speedup vs baseline: 1.0278x; 1.0278x over previous
import functools

import jax
import jax.numpy as jnp
from jax import lax
from jax.experimental import pallas as pl
from jax.experimental.pallas import tpu as pltpu
from jax.experimental.pallas import tpu_sc as plsc

_MAXR = 128
_ROWS = 2 * _MAXR + 1
_D = 1024
_B = 32 * 2048
_NC, _NS = 2, 16
_NW = _NC * _NS
_NG = 8
_ND = 4
_DCOL = _D // _ND
_KPR = _DCOL // 16
_RPG = _B // _NG
_RC = 64
_NBUF = 2
_LANES = 16


def _emb_body(idx_hbm, table_hbm, out_hbm, ttile, bufs, idx_v, *sems):
    tsem = sems[0]
    wsem = sems[1:]
    wid = lax.axis_index("s") * _NC + lax.axis_index("c")
    g = wid // _ND
    j = wid % _ND
    rbase = g * _RPG
    cbase = j * _DCOL

    isem = wsem[0]
    pltpu.async_copy(idx_hbm.at[pl.ds(rbase, _RPG)], idx_v, isem)

    def tload_start(v, carry):
        pltpu.async_copy(
            table_hbm.at[pl.ds(v * _D + cbase, _DCOL)],
            ttile.at[pl.ds(v * _DCOL, _DCOL)],
            tsem,
        )
        return carry

    lax.fori_loop(0, _ROWS, tload_start, 0)

    pltpu.make_async_copy(idx_hbm.at[pl.ds(rbase, _RPG)], idx_v, isem).wait()

    def clamp_body(i, carry):
        sl = pl.ds(i * _LANES, _LANES)
        v = idx_v[sl]
        idx_v[sl] = (
            jnp.minimum(jnp.maximum(v, -_MAXR), _MAXR) + _MAXR
        ) * _DCOL
        return carry

    lax.fori_loop(0, _RPG // _LANES, clamp_body, 0)

    def tload_wait(v, carry):
        pltpu.make_async_copy(
            table_hbm.at[pl.ds(v * _D + cbase, _DCOL)],
            ttile.at[pl.ds(v * _DCOL, _DCOL)],
            tsem,
        ).wait()
        return carry

    lax.fori_loop(0, _ROWS, tload_wait, 0)

    colvec = [lax.iota(jnp.int32, _LANES) + k * _LANES for k in range(_KPR)]
    _NCHW = _RPG // _RC
    _BLK = _RC // _LANES

    def outer_body(o, carry):
        for w in range(_NBUF):
            n = o * _NBUF + w
            row0 = rbase + n * _RC

            @pl.when(o > 0)
            def _wait_prev():
                pltpu.make_async_copy(
                    bufs.at[w],
                    out_hbm.at[pl.ds(row0 - _NBUF * _RC, _RC), pl.ds(cbase, _DCOL)],
                    wsem[w],
                ).wait()

            @plsc.parallel_loop(0, _RC, 1, unroll=4)
            def _rows(r):
                blk = r & ~(_LANES - 1)
                lane = r & (_LANES - 1)
                v16 = idx_v[pl.ds(n * _RC + blk, _LANES)]
                base = v16.at[jnp.full((_LANES,), lane, jnp.int32)].get(
                    mode="promise_in_bounds"
                )
                for k in range(_KPR):
                    val = plsc.load_gather(ttile, [base + colvec[k]])
                    bufs[w, r, pl.ds(k * _LANES, _LANES)] = val

            pltpu.async_copy(
                bufs.at[w],
                out_hbm.at[pl.ds(row0, _RC), pl.ds(cbase, _DCOL)],
                wsem[w],
            )
        return carry

    lax.fori_loop(0, _NCHW // _NBUF, outer_body, 0)

    for w in range(_NBUF):
        row0 = rbase + ((_NCHW // _NBUF - 1) * _NBUF + w) * _RC
        pltpu.make_async_copy(
            bufs.at[w],
            out_hbm.at[pl.ds(row0, _RC), pl.ds(cbase, _DCOL)],
            wsem[w],
        ).wait()


@jax.jit
def _emb_call(idx_flat, table_flat):
    mesh = plsc.VectorSubcoreMesh(core_axis_name="c", subcore_axis_name="s")
    fn = functools.partial(
        pl.kernel,
        mesh=mesh,
        compiler_params=pltpu.CompilerParams(needs_layout_passes=False),
        out_type=jax.ShapeDtypeStruct((_B, _D), jnp.float32),
        scratch_types=[
            pltpu.VMEM((_ROWS * _DCOL,), jnp.float32),
            pltpu.VMEM((_NBUF, _RC, _DCOL), jnp.float32),
            pltpu.VMEM((_RPG,), jnp.int32),
        ]
        + [pltpu.SemaphoreType.DMA] * (1 + _NBUF),
    )(_emb_body)
    return fn(idx_flat, table_flat)


def kernel(relative_dis, W):
    idx_flat = relative_dis.reshape(-1).astype(jnp.int32)
    out = _emb_call(idx_flat, W.reshape(-1))
    return out.reshape(relative_dis.shape + (_D,))
